# baseline (speedup 1.0000x reference)
import functools

import jax
import jax.numpy as jnp
from jax import lax
from jax.experimental import pallas as pl
from jax.experimental.pallas import tpu as pltpu
from jax.experimental.pallas import tpu_sc as plsc

EMBED = 64
NUM_CORES = 2
NUM_SUBCORES = 16
NW = NUM_CORES * NUM_SUBCORES
IDX_MINOR = 128
EPS = 1e-5


def _sc_gather(ids_flat, table):
    n_ids = ids_flat.shape[0]
    ids_per_w = n_ids // NW
    chunks_per_w = ids_per_w // IDX_MINOR
    mesh = plsc.VectorSubcoreMesh(
        core_axis_name="c", subcore_axis_name="s",
        num_cores=NUM_CORES, num_subcores=NUM_SUBCORES)

    @functools.partial(
        pl.kernel,
        out_type=jax.ShapeDtypeStruct((n_ids, EMBED), jnp.float32),
        mesh=mesh,
        scratch_types=[
            pltpu.VMEM((ids_per_w,), jnp.int32),
            pltpu.VMEM((IDX_MINOR, EMBED), jnp.float32),
            pltpu.SemaphoreType.DMA,
        ],
        compiler_params=pltpu.CompilerParams(use_tc_tiling_on_sc=False),
    )
    def k(ids_hbm, table_hbm, out_hbm, idx_v, rows_v, sem):
        wid = lax.axis_index("s") * NUM_CORES + lax.axis_index("c")
        base = wid * ids_per_w
        pltpu.sync_copy(ids_hbm.at[pl.ds(base, ids_per_w)], idx_v)

        def body(j, carry):
            pltpu.async_copy(
                table_hbm.at[idx_v.at[pl.ds(j * IDX_MINOR, IDX_MINOR)]],
                rows_v, sem).wait()
            pltpu.sync_copy(
                rows_v,
                out_hbm.at[pl.ds(base + j * IDX_MINOR, IDX_MINOR)])
            return carry

        lax.fori_loop(0, chunks_per_w, body, 0, unroll=False)

    return k(ids_flat, table)


def _ln_body(x_ref, pos_ref, gamma_ref, beta_ref, o_ref):
    x = x_ref[...] + pos_ref[...][None, :, :]
    mean = jnp.mean(x, axis=-1, keepdims=True)
    cent = x - mean
    var = jnp.mean(cent * cent, axis=-1, keepdims=True)
    xhat = cent * lax.rsqrt(var + EPS)
    o_ref[...] = xhat * gamma_ref[...][None, :] + beta_ref[...][None, :]


def _tc_ln(x, pos, gamma2d, beta2d):
    B, L, E = x.shape
    BB = 16
    return pl.pallas_call(
        _ln_body,
        grid=(B // BB,),
        in_specs=[
            pl.BlockSpec((BB, L, E), lambda i: (i, 0, 0)),
            pl.BlockSpec((L, E), lambda i: (0, 0)),
            pl.BlockSpec((1, E), lambda i: (0, 0)),
            pl.BlockSpec((1, E), lambda i: (0, 0)),
        ],
        out_specs=pl.BlockSpec((BB, L, E), lambda i: (i, 0, 0)),
        out_shape=jax.ShapeDtypeStruct((B, L, E), jnp.float32),
    )(x, pos, gamma2d, beta2d)


def kernel(input_ids, word_table, pos_table, gamma, beta):
    B, L = input_ids.shape
    gathered = _sc_gather(input_ids.reshape(-1), word_table)
    x = gathered.reshape(B, L, EMBED)
    return _tc_ln(x, pos_table[:L], gamma.reshape(1, EMBED),
                  beta.reshape(1, EMBED))


# speedup vs baseline: 1.1354x; 1.1354x over previous
import functools

import jax
import jax.numpy as jnp
from jax import lax
from jax.experimental import pallas as pl
from jax.experimental.pallas import tpu as pltpu
from jax.experimental.pallas import tpu_sc as plsc

EMBED = 64
PADE = 128
NUM_CORES = 2
NUM_SUBCORES = 16
NW = NUM_CORES * NUM_SUBCORES
CHUNK = 800
EPS = 1e-5


def _sc_gather(ids_flat, table):
    n_ids = ids_flat.shape[0]
    ids_per_w = n_ids // NW
    n_chunks = ids_per_w // CHUNK
    mesh = plsc.VectorSubcoreMesh(
        core_axis_name="c", subcore_axis_name="s",
        num_cores=NUM_CORES, num_subcores=NUM_SUBCORES)

    @functools.partial(
        pl.kernel,
        out_type=jax.ShapeDtypeStruct((n_ids, PADE), jnp.float32),
        mesh=mesh,
        scratch_types=[
            pltpu.VMEM((ids_per_w,), jnp.int32),
            pltpu.VMEM((2, CHUNK, EMBED), jnp.float32),
            pltpu.SemaphoreType.DMA,
            pltpu.SemaphoreType.DMA,
            pltpu.SemaphoreType.DMA,
            pltpu.SemaphoreType.DMA,
        ],
        compiler_params=pltpu.CompilerParams(use_tc_tiling_on_sc=False),
    )
    def k(ids_hbm, table_hbm, out_hbm, idx_v, rows_v, g0, g1, w0, w1):
        wid = lax.axis_index("s") * NUM_CORES + lax.axis_index("c")
        base = wid * ids_per_w
        pltpu.sync_copy(ids_hbm.at[pl.ds(base, ids_per_w)], idx_v)
        gsems = (g0, g1)
        wsems = (w0, w1)

        def gather(c):
            b = c % 2
            return pltpu.async_copy(
                table_hbm.at[idx_v.at[pl.ds(c * CHUNK, CHUNK)]],
                rows_v.at[b], gsems[b])

        def write(c):
            b = c % 2
            return pltpu.async_copy(
                rows_v.at[b],
                out_hbm.at[pl.ds(base + c * CHUNK, CHUNK), pl.ds(0, EMBED)],
                wsems[b])

        pending_w = [None, None]
        gather(0)
        for c in range(n_chunks):
            b = c % 2
            if c + 1 < n_chunks:
                if pending_w[1 - b] is not None:
                    pending_w[1 - b].wait()
                    pending_w[1 - b] = None
                gather(c + 1)
            pltpu.make_async_copy(
                table_hbm.at[idx_v.at[pl.ds(c * CHUNK, CHUNK)]],
                rows_v.at[b], gsems[b]).wait()
            pending_w[b] = write(c)
        for p in pending_w:
            if p is not None:
                p.wait()

    return k(ids_flat, table)


def _ln_body(x_ref, pos_ref, gamma_ref, beta_ref, o_ref):
    x = x_ref[:, :, :EMBED] + pos_ref[...][None, :, :]
    mean = jnp.mean(x, axis=-1, keepdims=True)
    cent = x - mean
    var = jnp.mean(cent * cent, axis=-1, keepdims=True)
    xhat = cent * lax.rsqrt(var + EPS)
    o_ref[...] = xhat * gamma_ref[...][None, :] + beta_ref[...][None, :]


def _tc_ln(x, pos, gamma2d, beta2d):
    B, L, E = x.shape
    BB = 16
    return pl.pallas_call(
        _ln_body,
        grid=(B // BB,),
        in_specs=[
            pl.BlockSpec((BB, L, E), lambda i: (i, 0, 0)),
            pl.BlockSpec((L, EMBED), lambda i: (0, 0)),
            pl.BlockSpec((1, EMBED), lambda i: (0, 0)),
            pl.BlockSpec((1, EMBED), lambda i: (0, 0)),
        ],
        out_specs=pl.BlockSpec((BB, L, EMBED), lambda i: (i, 0, 0)),
        out_shape=jax.ShapeDtypeStruct((B, L, EMBED), jnp.float32),
    )(x, pos, gamma2d, beta2d)


def kernel(input_ids, word_table, pos_table, gamma, beta):
    B, L = input_ids.shape
    gathered = _sc_gather(input_ids.reshape(-1), word_table)
    x = gathered.reshape(B, L, PADE)
    return _tc_ln(x, pos_table[:L], gamma.reshape(1, EMBED),
                  beta.reshape(1, EMBED))
